# baseline (speedup 1.0000x reference)
import functools

import numpy as np
import jax
import jax.numpy as jnp
from jax import lax
from jax.experimental import pallas as pl
from jax.experimental.pallas import tpu as pltpu
from jax.experimental.pallas import tpu_sc as plsc

_H = 768
_NT = 18
_NTP = 32
_W = _H + 128
_MAX_SPAN = 8
_B, _S = 4, 512
_R = 256
_NTILES = 32
_CHUNK = 32


def _span_index_lists():
    starts = np.repeat(np.arange(_S), _MAX_SPAN)
    ends = starts + np.tile(np.arange(_MAX_SPAN), _S)
    valid = ends < _S
    starts, ends = starts[valid], ends[valid]
    n = int(starts.size)
    idx_s = (np.arange(_B)[:, None] * _S + starts[None, :]).reshape(-1)
    idx_e = (np.arange(_B)[:, None] * _S + ends[None, :]).reshape(-1)
    total = -(-_B * n // (_NTILES * _CHUNK)) * (_NTILES * _CHUNK)
    pad = total - idx_s.size
    idx_s = np.concatenate([idx_s, np.zeros(pad, idx_s.dtype)])
    idx_e = np.concatenate([idx_e, np.zeros(pad, idx_e.dtype)])
    return idx_s.astype(np.int32), idx_e.astype(np.int32), n, total


_IDX_S_NP, _IDX_E_NP, _NSP, _TOT = _span_index_lists()
_SPT = _TOT // _NTILES


def _proj_body(hs_ref, wa_ref, wb_ref, wes_ref, wee_ref, wbd_ref,
               bs1_ref, be2_ref, bbd_ref,
               ts_ref, te_ref, bnd_ref):
    hs = hs_ref[...]
    dot = lambda a, b: jnp.dot(a, b, preferred_element_type=jnp.float32)
    ts_ref[:, : _H] = dot(hs, wa_ref[...]) + bs1_ref[...]
    ts_ref[:, _H:] = dot(hs, wes_ref[...]) + be2_ref[...]
    te_ref[:, : _H] = dot(hs, wb_ref[...])
    te_ref[:, _H:] = dot(hs, wee_ref[...]) + be2_ref[...]
    bnd_ref[...] = dot(hs, wbd_ref[...]) + bbd_ref[...]


def _proj_call(hs, wa, wb, wes, wee, wbd, bs1, be2, bbd):
    nrows = hs.shape[0]
    grid = (nrows // _R,)
    full = lambda shape: pl.BlockSpec(shape, lambda i: (0, 0))
    rows = lambda width: pl.BlockSpec((_R, width), lambda i: (i, 0))
    return pl.pallas_call(
        _proj_body,
        grid=grid,
        in_specs=[
            rows(_H),
            full((_H, _H)), full((_H, _H)),
            full((_H, 128)), full((_H, 128)), full((_H, 8)),
            full((1, _H)), full((1, 128)), full((1, 8)),
        ],
        out_specs=[rows(_W), rows(_W), rows(8)],
        out_shape=[
            jax.ShapeDtypeStruct((nrows, _W), jnp.float32),
            jax.ShapeDtypeStruct((nrows, _W), jnp.float32),
            jax.ShapeDtypeStruct((nrows, 8), jnp.float32),
        ],
    )(hs, wa, wb, wes, wee, wbd, bs1, be2, bbd)


@functools.lru_cache(maxsize=1)
def _span_call():
    mesh = plsc.VectorSubcoreMesh(core_axis_name="c", subcore_axis_name="s",
                                  num_cores=2, num_subcores=16)

    @functools.partial(
        pl.kernel,
        out_type=[
            jax.ShapeDtypeStruct((_TOT,), jnp.float32),
            jax.ShapeDtypeStruct((_TOT, _NTP), jnp.float32),
        ],
        mesh=mesh,
        compiler_params=pltpu.CompilerParams(needs_layout_passes=False),
        scratch_types=[
            pltpu.VMEM((_CHUNK,), jnp.int32),
            pltpu.VMEM((_CHUNK,), jnp.int32),
            pltpu.VMEM((_CHUNK, _W), jnp.float32),
            pltpu.VMEM((_CHUNK, _W), jnp.float32),
            pltpu.VMEM((_CHUNK, _NTP), jnp.float32),
            pltpu.VMEM((_CHUNK,), jnp.float32),
            pltpu.VMEM((_H,), jnp.float32),
            pltpu.VMEM((16,), jnp.float32),
            pltpu.SemaphoreType.DMA,
        ],
    )
    def span_kernel(ts_hbm, te_hbm, idxs_hbm, idxe_hbm,
                    wpar_hbm, out_sc_hbm, out_ent_hbm,
                    idxs_v, idxe_v, a_v, b_v, ent_v, sc_v, w_v, bi_v, sem):
        wid = lax.axis_index("s") * 2 + lax.axis_index("c")
        pltpu.sync_copy(wpar_hbm.at[pl.ds(0, _H)], w_v)
        pltpu.sync_copy(wpar_hbm.at[pl.ds(_H, 16)], bi_v)
        base0 = wid * _SPT

        def chunk_body(c, carry):
            base = base0 + c * _CHUNK
            pltpu.sync_copy(idxs_hbm.at[pl.ds(base, _CHUNK)], idxs_v)
            pltpu.sync_copy(idxe_hbm.at[pl.ds(base, _CHUNK)], idxe_v)
            pltpu.async_copy(ts_hbm.at[idxs_v], a_v, sem).wait()
            pltpu.async_copy(te_hbm.at[idxe_v], b_v, sem).wait()

            lane = lax.iota(jnp.int32, 16)

            def group_body(g, carry2):
                sp = g * 16 + lane

                def h_body(h, acc):
                    hvec = jnp.full((16,), h, jnp.int32)
                    va = plsc.load_gather(a_v, [sp, hvec])
                    vb = plsc.load_gather(b_v, [sp, hvec])
                    vw = plsc.load_gather(w_v, [hvec])
                    return acc + jnp.maximum(va + vb, 0.0) * vw

                acc = lax.fori_loop(0, _H, h_body, bi_v[...])
                sc_v[pl.ds(g * 16, 16)] = acc

                def ent_body(j, carry3):
                    i = g * 16 + j
                    lo, hi = pl.ds(0, 16), pl.ds(16, 16)
                    slo, shi = pl.ds(_H, 16), pl.ds(_H + 16, 16)
                    ent_v[i, lo] = a_v[i, slo] + b_v[i, slo]
                    ent_v[i, hi] = a_v[i, shi] + b_v[i, shi]
                    return carry3

                lax.fori_loop(0, 16, ent_body, 0)
                return carry2

            lax.fori_loop(0, _CHUNK // 16, group_body, 0)
            pltpu.sync_copy(sc_v, out_sc_hbm.at[pl.ds(base, _CHUNK)])
            pltpu.sync_copy(ent_v, out_ent_hbm.at[pl.ds(base, _CHUNK)])
            return carry

        lax.fori_loop(0, _SPT // _CHUNK, chunk_body, 0)

    return span_kernel


def kernel(hidden_states, attention_mask, W_b, b_b, W_e, b_e,
           W_s1, b_s1, W_s2, b_s2):
    del attention_mask
    f32 = jnp.float32
    hs = hidden_states.reshape(_B * _S, _H)

    wa = W_s1[:, :_H].T
    wb = W_s1[:, _H:].T
    wes = jnp.zeros((_H, 128), f32).at[:, :_NT].set(W_e[:, :_H].T)
    wee = jnp.zeros((_H, 128), f32).at[:, :_NT].set(W_e[:, _H:].T)
    wbd = jnp.zeros((_H, 8), f32).at[:, :3].set(W_b.T)
    bs1 = b_s1.reshape(1, _H)
    be2 = jnp.zeros((1, 128), f32).at[0, :_NT].set(0.5 * b_e)
    bbd = jnp.zeros((1, 8), f32).at[0, :3].set(b_b)

    ts, te, bnd = _proj_call(hs, wa, wb, wes, wee, wbd, bs1, be2, bbd)

    wpar = jnp.concatenate([W_s2[0], jnp.broadcast_to(b_s2, (16,))])
    idx_s = jnp.asarray(_IDX_S_NP)
    idx_e = jnp.asarray(_IDX_E_NP)
    scores_flat, ent_pad = _span_call()(ts, te, idx_s, idx_e, wpar)

    boundary_logits = bnd[:, :3].reshape(_B, _S, 3)
    span_scores = scores_flat[: _B * _NSP].reshape(_B, _NSP, 1)
    entity_logits = ent_pad[: _B * _NSP, :_NT].reshape(_B, _NSP, _NT)
    return boundary_logits, span_scores, entity_logits


# speedup vs baseline: 4.7679x; 4.7679x over previous
import functools

import numpy as np
import jax
import jax.numpy as jnp
from jax import lax
from jax.experimental import pallas as pl
from jax.experimental.pallas import tpu as pltpu
from jax.experimental.pallas import tpu_sc as plsc

_H = 768
_NT = 18
_OROW = 128
_SCOL = 18
_W = _H + 128
_MAX_SPAN = 8
_B, _S = 4, 512
_NSP = 4068
_R = 256
_TROWS = 2304
_NTILES = 32
_TSTARTS = 64
_SUB = 16
_NSUBS = _TSTARTS // _SUB
_BLK = _SUB * _MAX_SPAN
_TOT = _B * _S * _MAX_SPAN


def _dest_indices():
    dest = np.zeros((_B, _S, _MAX_SPAN), np.int64)
    dump = _B * _NSP
    for b in range(_B):
        pos = b * _NSP
        for s in range(_S):
            for k in range(_MAX_SPAN):
                if s + k < _S:
                    dest[b, s, k] = pos
                    pos += 1
                else:
                    dest[b, s, k] = dump
                    dump += 1
    didx = np.zeros(_TOT, np.int64)
    p = 0
    for wid in range(_NTILES):
        b, tb = wid // 8, wid % 8
        for s_local in range(_TSTARTS):
            s = tb * _TSTARTS + s_local
            for k in range(_MAX_SPAN):
                didx[p] = dest[b, s, k]
                p += 1
    return didx.astype(np.int32)


_DIDX_NP = _dest_indices()


def _proj_body(hs_ref, wa_ref, wb_ref, wes_ref, wee_ref, wbd_ref,
               bs1_ref, be2_ref, bbd_ref,
               ts_ref, te_ref, bnd_ref):
    hs = hs_ref[...]
    dot = lambda a, b: jnp.dot(a, b, preferred_element_type=jnp.float32)
    ts_ref[:, : _H] = dot(hs, wa_ref[...]) + bs1_ref[...]
    ts_ref[:, _H:] = dot(hs, wes_ref[...]) + be2_ref[...]
    te_ref[:, : _H] = dot(hs, wb_ref[...])
    te_ref[:, _H:] = dot(hs, wee_ref[...]) + be2_ref[...]
    bnd_ref[...] = dot(hs, wbd_ref[...]) + bbd_ref[...]


def _proj_call(hs, wa, wb, wes, wee, wbd, bs1, be2, bbd):
    nrows = hs.shape[0]
    grid = (nrows // _R,)
    full = lambda shape: pl.BlockSpec(shape, lambda i: (0, 0))
    rows = lambda width: pl.BlockSpec((_R, width), lambda i: (i, 0))
    return pl.pallas_call(
        _proj_body,
        grid=grid,
        in_specs=[
            rows(_H),
            full((_H, _H)), full((_H, _H)),
            full((_H, 128)), full((_H, 128)), full((_H, 8)),
            full((1, _H)), full((1, 128)), full((1, 8)),
        ],
        out_specs=[rows(_W), rows(_W), rows(8)],
        out_shape=[
            jax.ShapeDtypeStruct((_TROWS, _W), jnp.float32),
            jax.ShapeDtypeStruct((_TROWS, _W), jnp.float32),
            jax.ShapeDtypeStruct((nrows, 8), jnp.float32),
        ],
    )(hs, wa, wb, wes, wee, wbd, bs1, be2, bbd)


@functools.lru_cache(maxsize=1)
def _span_call():
    mesh = plsc.VectorSubcoreMesh(core_axis_name="c", subcore_axis_name="s",
                                  num_cores=2, num_subcores=16)

    @functools.partial(
        pl.kernel,
        out_type=jax.ShapeDtypeStruct((_TOT, _OROW), jnp.float32),
        mesh=mesh,
        compiler_params=pltpu.CompilerParams(needs_layout_passes=False),
        scratch_types=[
            pltpu.VMEM((_SUB, _W), jnp.float32),
            pltpu.VMEM((_SUB + _MAX_SPAN, _W), jnp.float32),
            pltpu.VMEM((_BLK, _OROW), jnp.float32),
            pltpu.VMEM((_BLK,), jnp.int32),
            pltpu.VMEM((_H,), jnp.float32),
            pltpu.VMEM((16,), jnp.float32),
            pltpu.SemaphoreType.DMA,
        ],
    )
    def span_kernel(ts_hbm, te_hbm, didx_hbm, wpar_hbm, out_hbm,
                    a_v, b_v, o_v, didx_v, w_v, bi_v, sem):
        wid = lax.axis_index("s") * 2 + lax.axis_index("c")
        pltpu.sync_copy(wpar_hbm.at[pl.ds(0, _H)], w_v)
        pltpu.sync_copy(wpar_hbm.at[pl.ds(_H, 16)], bi_v)
        row0 = (wid // 8) * _S + (wid % 8) * _TSTARTS
        obase0 = wid * _TSTARTS * _MAX_SPAN
        lane = lax.iota(jnp.int32, 16)
        zero16 = jnp.zeros((16,), jnp.float32)

        def sub_body(sub, carry):
            r0 = row0 + sub * _SUB
            obase = obase0 + sub * _BLK
            pltpu.sync_copy(ts_hbm.at[pl.ds(r0, _SUB)], a_v)
            pltpu.sync_copy(te_hbm.at[pl.ds(r0, _SUB + _MAX_SPAN)], b_v)
            pltpu.sync_copy(didx_hbm.at[pl.ds(obase, _BLK)], didx_v)

            def start_body(i, carry2):
                def hb_body(hb, accs):
                    sl = pl.ds(hb * 16, 16)
                    va = a_v[i, sl]
                    vw = w_v[sl]
                    return tuple(
                        accs[k] + jnp.maximum(va + b_v[i + k, sl], 0.0) * vw
                        for k in range(_MAX_SPAN)
                    )

                accs = lax.fori_loop(0, _H // 16, hb_body,
                                     (zero16,) * _MAX_SPAN)
                lo, hi = pl.ds(_H, 16), pl.ds(_H + 16, 16)
                ealo = a_v[i, lo]
                eahi = a_v[i, hi]
                for k in range(_MAX_SPAN):
                    x = accs[k]
                    for sh in (8, 4, 2, 1):
                        x = x + jnp.take_along_axis(
                            x, lane ^ sh, axis=0, mode="promise_in_bounds")
                    x = x + bi_v[...]
                    r = i * _MAX_SPAN + k
                    o_v[r, pl.ds(0, 16)] = ealo + b_v[i + k, lo]
                    o_v[r, pl.ds(16, 16)] = jnp.where(
                        lane == _SCOL - 16, x, eahi + b_v[i + k, hi])
                return carry2

            lax.fori_loop(0, _SUB, start_body, 0)
            pltpu.async_copy(o_v, out_hbm.at[didx_v], sem).wait()
            return carry

        lax.fori_loop(0, _NSUBS, sub_body, 0)

    return span_kernel


def kernel(hidden_states, attention_mask, W_b, b_b, W_e, b_e,
           W_s1, b_s1, W_s2, b_s2):
    del attention_mask
    f32 = jnp.float32
    hs = hidden_states.reshape(_B * _S, _H)

    wa = W_s1[:, :_H].T
    wb = W_s1[:, _H:].T
    wes = jnp.zeros((_H, 128), f32).at[:, :_NT].set(W_e[:, :_H].T)
    wee = jnp.zeros((_H, 128), f32).at[:, :_NT].set(W_e[:, _H:].T)
    wbd = jnp.zeros((_H, 8), f32).at[:, :3].set(W_b.T)
    bs1 = b_s1.reshape(1, _H)
    be2 = jnp.zeros((1, 128), f32).at[0, :_NT].set(0.5 * b_e)
    bbd = jnp.zeros((1, 8), f32).at[0, :3].set(b_b)

    ts, te, bnd = _proj_call(hs, wa, wb, wes, wee, wbd, bs1, be2, bbd)

    wpar = jnp.concatenate([W_s2[0], jnp.broadcast_to(b_s2, (16,))])
    out = _span_call()(ts, te, jnp.asarray(_DIDX_NP), wpar)

    boundary_logits = bnd[:, :3].reshape(_B, _S, 3)
    span_scores = out[: _B * _NSP, _SCOL].reshape(_B, _NSP, 1)
    entity_logits = out[: _B * _NSP, :_NT].reshape(_B, _NSP, _NT)
    return boundary_logits, span_scores, entity_logits


# speedup vs baseline: 6.3470x; 1.3312x over previous
import functools

import numpy as np
import jax
import jax.numpy as jnp
from jax import lax
from jax.experimental import pallas as pl
from jax.experimental.pallas import tpu as pltpu
from jax.experimental.pallas import tpu_sc as plsc

_H = 768
_NT = 18
_OROW = 128
_SCOL = 18
_W = _H + 128
_MAX_SPAN = 8
_B, _S = 4, 512
_NSP = 4068
_R = 256
_TROWS = 2304
_NTILES = 32
_TSTARTS = 64
_SUB = 16
_NSUBS = _TSTARTS // _SUB
_BLK = _SUB * _MAX_SPAN
_TOT = _B * _S * _MAX_SPAN
_HB_UNROLL = 3


def _dest_indices():
    dest = np.zeros((_B, _S, _MAX_SPAN), np.int64)
    dump = _B * _NSP
    for b in range(_B):
        pos = b * _NSP
        for s in range(_S):
            for k in range(_MAX_SPAN):
                if s + k < _S:
                    dest[b, s, k] = pos
                    pos += 1
                else:
                    dest[b, s, k] = dump
                    dump += 1
    didx = np.zeros(_TOT, np.int64)
    p = 0
    for wid in range(_NTILES):
        b, tb = wid // 8, wid % 8
        for s_local in range(_TSTARTS):
            s = tb * _TSTARTS + s_local
            for k in range(_MAX_SPAN):
                didx[p] = dest[b, s, k]
                p += 1
    return didx.astype(np.int32)


_DIDX_NP = _dest_indices()


def _proj_body(hs_ref, ws1_ref, we_ref, wb_ref,
               bs1_ref, be2_ref, bbd_ref,
               ts_ref, te_ref, bnd_ref):
    hs = hs_ref[...]
    w1 = ws1_ref[...]
    we = we_ref[...]
    dn = (((1,), (1,)), ((), ()))
    dot = lambda a, b: lax.dot_general(a, b, dn,
                                       preferred_element_type=jnp.float32)
    ts_ref[:, : _H] = dot(hs, w1[:, :_H]) + bs1_ref[...]
    ts_ref[:, _H : _H + _NT] = dot(hs, we[:, :_H]) + be2_ref[...]
    te_ref[:, : _H] = dot(hs, w1[:, _H:])
    te_ref[:, _H : _H + _NT] = dot(hs, we[:, _H:]) + be2_ref[...]
    bnd_ref[:, :3] = dot(hs, wb_ref[...]) + bbd_ref[...]


def _proj_call(hs, ws1, we, wb, bs1, be2, bbd):
    nrows = hs.shape[0]
    grid = (nrows // _R,)
    full = lambda shape: pl.BlockSpec(shape, lambda i: (0, 0))
    rows = lambda width: pl.BlockSpec((_R, width), lambda i: (i, 0))
    return pl.pallas_call(
        _proj_body,
        grid=grid,
        in_specs=[
            rows(_H),
            full((_H, 2 * _H)), full((_NT, 2 * _H)), full((3, _H)),
            full((1, _H)), full((1, _NT)), full((1, 3)),
        ],
        out_specs=[rows(_W), rows(_W), rows(8)],
        out_shape=[
            jax.ShapeDtypeStruct((_TROWS, _W), jnp.float32),
            jax.ShapeDtypeStruct((_TROWS, _W), jnp.float32),
            jax.ShapeDtypeStruct((nrows, 8), jnp.float32),
        ],
    )(hs, ws1, we, wb, bs1, be2, bbd)


@functools.lru_cache(maxsize=1)
def _span_call():
    mesh = plsc.VectorSubcoreMesh(core_axis_name="c", subcore_axis_name="s",
                                  num_cores=2, num_subcores=16)

    @functools.partial(
        pl.kernel,
        out_type=jax.ShapeDtypeStruct((_TOT, _OROW), jnp.float32),
        mesh=mesh,
        compiler_params=pltpu.CompilerParams(needs_layout_passes=False),
        scratch_types=[
            pltpu.VMEM((2, _SUB, _W), jnp.float32),
            pltpu.VMEM((2, _SUB + _MAX_SPAN, _W), jnp.float32),
            pltpu.VMEM((2, _BLK, _OROW), jnp.float32),
            pltpu.VMEM((2, _BLK), jnp.int32),
            pltpu.VMEM((_H,), jnp.float32),
            pltpu.VMEM((16,), jnp.float32),
            pltpu.SemaphoreType.DMA,
            pltpu.SemaphoreType.DMA,
            pltpu.SemaphoreType.DMA,
            pltpu.SemaphoreType.DMA,
        ],
    )
    def span_kernel(ts_hbm, te_hbm, didx_hbm, wpar_hbm, out_hbm,
                    a_v, b_v, o_v, didx_v, w_v, bi_v,
                    sema, semb, semd, semo):
        wid = lax.axis_index("s") * 2 + lax.axis_index("c")
        pltpu.sync_copy(wpar_hbm.at[pl.ds(0, _H)], w_v)
        pltpu.sync_copy(wpar_hbm.at[pl.ds(_H, 16)], bi_v)
        row0 = (wid // 8) * _S + (wid % 8) * _TSTARTS
        obase0 = wid * _TSTARTS * _MAX_SPAN
        lane = lax.iota(jnp.int32, 16)
        zero16 = jnp.zeros((16,), jnp.float32)

        def issue_in(sub):
            slot = sub % 2
            r0 = row0 + sub * _SUB
            obase = obase0 + sub * _BLK
            ca = pltpu.async_copy(ts_hbm.at[pl.ds(r0, _SUB)],
                                  a_v.at[slot], sema)
            cb = pltpu.async_copy(te_hbm.at[pl.ds(r0, _SUB + _MAX_SPAN)],
                                  b_v.at[slot], semb)
            cd = pltpu.async_copy(didx_hbm.at[pl.ds(obase, _BLK)],
                                  didx_v.at[slot], semd)
            return ca, cb, cd

        pend_in = issue_in(0)
        pend_out = [None, None]
        for sub in range(_NSUBS):
            slot = sub % 2
            for c in pend_in:
                c.wait()
            if sub + 1 < _NSUBS:
                nslot = (sub + 1) % 2
                if pend_out[nslot] is not None:
                    pend_out[nslot].wait()
                    pend_out[nslot] = None
                pend_in = issue_in(sub + 1)
            a_s = a_v.at[slot]
            b_s = b_v.at[slot]
            o_s = o_v.at[slot]
            if pend_out[slot] is not None:
                pend_out[slot].wait()
                pend_out[slot] = None

            def start_body(i, carry2, a_s=a_s, b_s=b_s, o_s=o_s):
                def hb_body(t, accs):
                    for u in range(_HB_UNROLL):
                        hb = t * _HB_UNROLL + u
                        sl = pl.ds(hb * 16, 16)
                        va = a_s[i, sl]
                        vw = w_v[sl]
                        accs = tuple(
                            accs[k] + jnp.maximum(va + b_s[i + k, sl], 0.0)
                            * vw
                            for k in range(_MAX_SPAN)
                        )
                    return accs

                accs = lax.fori_loop(0, _H // 16 // _HB_UNROLL, hb_body,
                                     (zero16,) * _MAX_SPAN)
                lo, hi = pl.ds(_H, 16), pl.ds(_H + 16, 16)
                ealo = a_s[i, lo]
                eahi = a_s[i, hi]
                for k in range(_MAX_SPAN):
                    x = accs[k]
                    for sh in (8, 4, 2, 1):
                        x = x + jnp.take_along_axis(
                            x, lane ^ sh, axis=0, mode="promise_in_bounds")
                    x = x + bi_v[...]
                    r = i * _MAX_SPAN + k
                    o_s[r, pl.ds(0, 16)] = ealo + b_s[i + k, lo]
                    o_s[r, pl.ds(16, 16)] = jnp.where(
                        lane == _SCOL - 16, x, eahi + b_s[i + k, hi])
                return carry2

            lax.fori_loop(0, _SUB, start_body, 0)
            pend_out[slot] = pltpu.async_copy(
                o_s, out_hbm.at[didx_v.at[slot]], semo)
        for c in pend_out:
            if c is not None:
                c.wait()

    return span_kernel


def kernel(hidden_states, attention_mask, W_b, b_b, W_e, b_e,
           W_s1, b_s1, W_s2, b_s2):
    del attention_mask
    f32 = jnp.float32
    hs = hidden_states.reshape(_B * _S, _H)

    ts, te, bnd = _proj_call(hs, W_s1, W_e, W_b,
                             b_s1.reshape(1, _H),
                             (0.5 * b_e).reshape(1, _NT),
                             b_b.reshape(1, 3))

    wpar = jnp.concatenate([W_s2[0], jnp.broadcast_to(b_s2, (16,))])
    out = _span_call()(ts, te, jnp.asarray(_DIDX_NP), wpar)

    boundary_logits = bnd[:, :3].reshape(_B, _S, 3)
    span_scores = out[: _B * _NSP, _SCOL].reshape(_B, _NSP, 1)
    entity_logits = out[: _B * _NSP, :_NT].reshape(_B, _NSP, _NT)
    return boundary_logits, span_scores, entity_logits
